# baseline (speedup 1.0000x reference)
import functools

import jax
import jax.numpy as jnp
from jax import lax
from jax.experimental import pallas as pl
from jax.experimental.pallas import tpu as pltpu, tpu_sc as plsc

_V = 1000
_D = 16
_B = 1024
_VB = 100


def _fill_kernel(x_ref, w0_ref, w1_ref, o_ref):
    i = pl.program_id(0)
    xv = x_ref[...]
    viota = jax.lax.broadcasted_iota(jnp.int32, (_VB, _D, _B), 0) + i * _VB
    mask = viota == xv
    base = w0_ref[...]
    alt = w1_ref[...]
    o_ref[...] = jnp.where(mask, alt, base)


def _kernel_tc(x, W):
    x3 = x.astype(jnp.int32).reshape(1, 1, _B)
    w0 = W[0].reshape(1, _D, 1)
    w1 = W[1].reshape(1, _D, 1)
    out_t = pl.pallas_call(
        _fill_kernel,
        grid=(_V // _VB,),
        in_specs=[
            pl.BlockSpec((1, 1, _B), lambda i: (0, 0, 0)),
            pl.BlockSpec((1, _D, 1), lambda i: (0, 0, 0)),
            pl.BlockSpec((1, _D, 1), lambda i: (0, 0, 0)),
        ],
        out_specs=pl.BlockSpec((_VB, _D, _B), lambda i: (i, 0, 0)),
        out_shape=jax.ShapeDtypeStruct((_V, _D, _B), jnp.float32),
    )(x3, w0, w1)
    return jnp.transpose(out_t, (2, 0, 1))


_ROW = _V * _D


def _sc_body(base_hbm, x_hbm, w1_hbm, out_hbm, img_a, img_b, xv, w1v, sem_a, sem_b):
    info = plsc.get_sparse_core_info()
    nc, ns = info.num_cores, info.num_subcores
    nw = nc * ns
    bpw = _B // nw
    wid = lax.axis_index("s") * nc + lax.axis_index("c")

    pltpu.sync_copy(base_hbm, img_a)
    pltpu.sync_copy(base_hbm, img_b)
    pltpu.sync_copy(x_hbm.at[pl.ds(pl.multiple_of(wid * bpw, 8), bpw)], xv)
    pltpu.sync_copy(w1_hbm, w1v)

    lanes = lax.iota(jnp.int32, 16)
    dnums = lax.GatherDimensionNumbers(
        offset_dims=(), collapsed_slice_dims=(0,), start_index_map=(0,))
    w0vec = img_a[pl.ds(0, _D)]
    w1vec = w1v[...]

    def _splat(vec, j):
        idx = jnp.full((16, 1), j, jnp.int32)
        return lax.gather(vec, idx, dnums, (1,),
                          mode=lax.GatherScatterMode.PROMISE_IN_BOUNDS)

    handles = [None] * bpw
    xbs = [None] * bpw
    for b in range(bpw):
        i, j = divmod(b, 16)
        xg = xv[pl.ds(i * 16, 16)]
        xbvec = _splat(xg, j)
        img, sem = (img_a, sem_a) if b % 2 == 0 else (img_b, sem_b)
        if b >= 2:
            handles[b - 2].wait()
            plsc.store_scatter(img, [xbs[b - 2] * _D + lanes], w0vec)
        plsc.store_scatter(img, [xbvec * _D + lanes], w1vec)
        xbs[b] = xbvec
        off = pl.multiple_of((wid * bpw + b) * _ROW, 8)
        handles[b] = pltpu.async_copy(img, out_hbm.at[pl.ds(off, _ROW)], sem)
    handles[bpw - 2].wait()
    handles[bpw - 1].wait()


def _kernel_sc(x, W):
    base = jnp.tile(W[0], _V)
    w1 = W[1]
    xi = x.astype(jnp.int32)
    run = functools.partial(
        pl.kernel,
        out_type=jax.ShapeDtypeStruct((_B * _ROW,), jnp.float32),
        mesh=plsc.VectorSubcoreMesh(core_axis_name="c", subcore_axis_name="s"),
        compiler_params=pltpu.CompilerParams(use_tc_tiling_on_sc=False, needs_layout_passes=False),
        scratch_types=[
            pltpu.VMEM((_ROW,), jnp.float32),
            pltpu.VMEM((_ROW,), jnp.float32),
            pltpu.VMEM((32,), jnp.int32),
            pltpu.VMEM((_D,), jnp.float32),
            pltpu.SemaphoreType.DMA,
            pltpu.SemaphoreType.DMA,
        ],
    )(_sc_body)
    out = run(base, xi, w1)
    return out.reshape(_B, _V, _D)


def kernel(x, W):
    return _kernel_sc(x, W)


# speedup vs baseline: 25.8442x; 25.8442x over previous
import functools

import jax
import jax.numpy as jnp
from jax import lax
from jax.experimental import pallas as pl
from jax.experimental.pallas import tpu as pltpu, tpu_sc as plsc

_V = 1000
_D = 16
_B = 1024
_VB = 100


def _fill_kernel(x_ref, w0_ref, w1_ref, o_ref):
    i = pl.program_id(0)
    xv = x_ref[...]
    viota = jax.lax.broadcasted_iota(jnp.int32, (_VB, _D, _B), 0) + i * _VB
    mask = viota == xv
    base = w0_ref[...]
    alt = w1_ref[...]
    o_ref[...] = jnp.where(mask, alt, base)


def _kernel_tc(x, W):
    x3 = x.astype(jnp.int32).reshape(1, 1, _B)
    w0 = W[0].reshape(1, _D, 1)
    w1 = W[1].reshape(1, _D, 1)
    out_t = pl.pallas_call(
        _fill_kernel,
        grid=(_V // _VB,),
        in_specs=[
            pl.BlockSpec((1, 1, _B), lambda i: (0, 0, 0)),
            pl.BlockSpec((1, _D, 1), lambda i: (0, 0, 0)),
            pl.BlockSpec((1, _D, 1), lambda i: (0, 0, 0)),
        ],
        out_specs=pl.BlockSpec((_VB, _D, _B), lambda i: (i, 0, 0)),
        out_shape=jax.ShapeDtypeStruct((_V, _D, _B), jnp.float32),
    )(x3, w0, w1)
    return jnp.transpose(out_t, (2, 0, 1))


_ROW = _V * _D


def _sc_body(base_hbm, x_hbm, w1_hbm, out_hbm, img_a, img_b, xv, w1v, sem_a, sem_b):
    info = plsc.get_sparse_core_info()
    nc, ns = info.num_cores, info.num_subcores
    nw = nc * ns
    bpw = _B // nw
    wid = lax.axis_index("s") * nc + lax.axis_index("c")

    pltpu.sync_copy(base_hbm, img_a)
    pltpu.sync_copy(base_hbm, img_b)
    pltpu.sync_copy(x_hbm.at[pl.ds(pl.multiple_of(wid * bpw, 8), bpw)], xv)
    pltpu.sync_copy(w1_hbm, w1v)

    lanes = lax.iota(jnp.int32, 16)
    dnums = lax.GatherDimensionNumbers(
        offset_dims=(), collapsed_slice_dims=(0,), start_index_map=(0,))
    w0vec = img_a[pl.ds(0, _D)]
    w1vec = w1v[...]

    def _splat(vec, j):
        idx = jnp.full((16, 1), j, jnp.int32)
        return lax.gather(vec, idx, dnums, (1,),
                          mode=lax.GatherScatterMode.PROMISE_IN_BOUNDS)

    handles = [None] * bpw
    xbs = [None] * bpw
    for b in range(bpw):
        i, j = divmod(b, 16)
        xg = xv[pl.ds(i * 16, 16)]
        xbvec = _splat(xg, j)
        img, sem = (img_a, sem_a) if b % 2 == 0 else (img_b, sem_b)
        if b >= 2:
            handles[b - 2].wait()
            plsc.store_scatter(img, [xbs[b - 2] * _D + lanes], w0vec)
        plsc.store_scatter(img, [xbvec * _D + lanes], w1vec)
        xbs[b] = xbvec
        off = pl.multiple_of((wid * bpw + b) * _ROW, 8)
        handles[b] = pltpu.async_copy(img, out_hbm.at[pl.ds(off, _ROW)], sem)
    handles[bpw - 2].wait()
    handles[bpw - 1].wait()


def _kernel_sc(x, W):
    base = jnp.tile(W[0], _V)
    w1 = W[1]
    xi = x.astype(jnp.int32)
    run = functools.partial(
        pl.kernel,
        out_type=jax.ShapeDtypeStruct((_B * _ROW,), jnp.float32),
        mesh=plsc.VectorSubcoreMesh(core_axis_name="c", subcore_axis_name="s"),
        compiler_params=pltpu.CompilerParams(use_tc_tiling_on_sc=False, needs_layout_passes=False),
        scratch_types=[
            pltpu.VMEM((_ROW,), jnp.float32),
            pltpu.VMEM((_ROW,), jnp.float32),
            pltpu.VMEM((32,), jnp.int32),
            pltpu.VMEM((_D,), jnp.float32),
            pltpu.SemaphoreType.DMA,
            pltpu.SemaphoreType.DMA,
        ],
    )(_sc_body)
    out = run(base, xi, w1)
    return out.reshape(_B, _V, _D)


def kernel(x, W):
    return _kernel_tc(x, W)
